# Initial kernel scaffold; baseline (speedup 1.0000x reference)
import jax
import jax.numpy as jnp
from jax.experimental import pallas as pl


def kernel(x, edge_index, edge_attr, u, batch, W1, b1, W2, b2):
    raise NotImplementedError("write your pallas kernel here")


# speedup vs baseline: 13.6905x; 13.6905x over previous
import functools

import jax
import jax.numpy as jnp
from jax import lax
from jax.experimental import pallas as pl
from jax.experimental.pallas import tpu as pltpu
from jax.experimental.pallas import tpu_sc as plsc

N = 100000
NUM_SEG = 512
NC = 2
NS = 16
NW = NC * NS
CHUNK = 3136
NPAD = NW * CHUNK
STEPS = CHUNK // 16
SEGP = 528
COLS = SEGP // 16


def _sc_segment_sums(x0, x1, batch):
    mesh = plsc.VectorSubcoreMesh(core_axis_name="c", subcore_axis_name="s")

    @functools.partial(
        pl.kernel,
        mesh=mesh,
        compiler_params=pltpu.CompilerParams(needs_layout_passes=False),
        out_type=[jax.ShapeDtypeStruct((NW, SEGP), jnp.float32)] * 3,
        scratch_types=[
            pltpu.VMEM((CHUNK,), jnp.float32),
            pltpu.VMEM((CHUNK,), jnp.float32),
            pltpu.VMEM((CHUNK,), jnp.int32),
            pltpu.VMEM((NS * SEGP,), jnp.float32),
            pltpu.VMEM((NS * SEGP,), jnp.float32),
            pltpu.VMEM((NS * SEGP,), jnp.float32),
            pltpu.VMEM((SEGP,), jnp.float32),
            pltpu.VMEM((SEGP,), jnp.float32),
            pltpu.VMEM((SEGP,), jnp.float32),
        ],
    )
    def k(x0_hbm, x1_hbm, b_hbm, out0, out1, outc,
          x0v, x1v, bv, acc0, acc1, accc, st0, st1, stc):
        wid = lax.axis_index("s") * NC + lax.axis_index("c")
        base = wid * CHUNK
        pltpu.sync_copy(x0_hbm.at[pl.ds(base, CHUNK)], x0v)
        pltpu.sync_copy(x1_hbm.at[pl.ds(base, CHUNK)], x1v)
        pltpu.sync_copy(b_hbm.at[pl.ds(base, CHUNK)], bv)

        zeros = jnp.zeros((16,), jnp.float32)
        ones = jnp.ones((16,), jnp.float32)
        laneoff = lax.iota(jnp.int32, 16) * SEGP

        def zero_body(c, carry):
            off = c * 16
            for r in range(NS):
                acc0[pl.ds(off + r * SEGP, 16)] = zeros
                acc1[pl.ds(off + r * SEGP, 16)] = zeros
                accc[pl.ds(off + r * SEGP, 16)] = zeros
            return carry

        lax.fori_loop(0, COLS, zero_body, 0)

        def body(i, carry):
            off = i * 16
            tgt = laneoff + bv[pl.ds(off, 16)]
            v0 = x0v[pl.ds(off, 16)]
            v1 = x1v[pl.ds(off, 16)]
            plsc.addupdate_scatter(acc0, [tgt], v0)
            plsc.addupdate_scatter(acc1, [tgt], v1)
            plsc.addupdate_scatter(accc, [tgt], ones)
            return carry

        lax.fori_loop(0, STEPS, body, 0)

        def red_body(c, carry):
            off = c * 16
            s0 = acc0[pl.ds(off, 16)]
            s1 = acc1[pl.ds(off, 16)]
            sc = accc[pl.ds(off, 16)]
            for r in range(1, NS):
                s0 = s0 + acc0[pl.ds(off + r * SEGP, 16)]
                s1 = s1 + acc1[pl.ds(off + r * SEGP, 16)]
                sc = sc + accc[pl.ds(off + r * SEGP, 16)]
            st0[pl.ds(off, 16)] = s0
            st1[pl.ds(off, 16)] = s1
            stc[pl.ds(off, 16)] = sc
            return carry

        lax.fori_loop(0, COLS, red_body, 0)

        pltpu.sync_copy(st0, out0.at[wid])
        pltpu.sync_copy(st1, out1.at[wid])
        pltpu.sync_copy(stc, outc.at[wid])

    return k(x0, x1, batch)


def _tc_mean_mlp(p0, p1, pc, W1, b1, W2, b2):
    def body(p0_ref, p1_ref, pc_ref, w1_ref, b1_ref, w2_ref, b2_ref, out_ref):
        s0 = jnp.sum(p0_ref[...], axis=0)[:NUM_SEG]
        s1 = jnp.sum(p1_ref[...], axis=0)[:NUM_SEG]
        cnt = jnp.sum(pc_ref[...], axis=0)[:NUM_SEG]
        denom = jnp.maximum(cnt, 1.0)
        m0 = (s0 / denom)[:, None]
        m1 = (s1 / denom)[:, None]
        w1 = w1_ref[...]
        h = m0 * w1[0:1, :] + m1 * w1[1:2, :] + b1_ref[...][None, :]
        h = jnp.maximum(h, 0.0)
        out_ref[...] = (
            jnp.dot(h, w2_ref[...], preferred_element_type=jnp.float32)
            + b2_ref[...][None, :]
        )

    return pl.pallas_call(
        body,
        out_shape=jax.ShapeDtypeStruct((NUM_SEG, 32), jnp.float32),
    )(p0, p1, pc, W1, b1, W2, b2)


def kernel(x, edge_index, edge_attr, u, batch, W1, b1, W2, b2):
    del edge_index, edge_attr, u
    x0 = jnp.pad(x[:, 0], (0, NPAD - N))
    x1 = jnp.pad(x[:, 1], (0, NPAD - N))
    b = jnp.pad(batch.astype(jnp.int32), (0, NPAD - N),
                constant_values=NUM_SEG)
    p0, p1, pc = _sc_segment_sums(x0, x1, b)
    return _tc_mean_mlp(p0, p1, pc, W1, b1, W2, b2)
